# baseline (speedup 1.0000x reference)
import functools

import jax
import jax.numpy as jnp
from jax.experimental import pallas as pl
from jax.experimental.pallas import tpu as pltpu

_PARITIES = ((0, 0), (0, 1), (1, 0), (1, 1))
_TAP = {(0, 0): 3, (0, 1): 1, (1, 1): 2, (1, 2): 0}


def _bn_scale_shift(col_sum, col_sumsq, count, gamma, beta, eps=1e-5):
    mean = col_sum / count
    var = jnp.maximum(col_sumsq / count - mean * mean, 0.0)
    scale = gamma * jax.lax.rsqrt(var + eps)
    shift = beta - mean * scale
    return scale, shift


def _parity_weights(w_t):
    Cin, Cout = w_t.shape[0], w_t.shape[1]
    w16 = w_t.astype(jnp.bfloat16).transpose(2, 3, 0, 1).reshape(
        16, Cin, Cout)
    idx = []
    for rh, rw in _PARITIES:
        dhs = (0, 1) if rh == 0 else (1, 2)
        dws = (0, 1) if rw == 0 else (1, 2)
        idx += [_TAP[(rh, dh)] * 4 + _TAP[(rw, dw)]
                for dh in dhs for dw in dws]
    return w16[jnp.array(idx)].reshape(4, 2, 2 * Cin, Cout)


def _dense_deconv_weights(w_t):
    Cin, Cout = w_t.shape[0], w_t.shape[1]
    w16 = w_t.astype(jnp.bfloat16).transpose(2, 3, 0, 1).reshape(
        16, Cin, Cout)
    w17 = jnp.concatenate([w16, jnp.zeros((1, Cin, Cout), jnp.bfloat16)])
    idx = [[_TAP[(rh, dh)] * 4 + _TAP[(rw, dw)]
            if ((rh, dh) in _TAP and (rw, dw) in _TAP) else 16
            for rh, rw in _PARITIES]
           for dh in range(3) for dw in range(3)]
    sel = w17[jnp.array(idx)].transpose(0, 2, 1, 3)
    return sel.reshape(3, 3 * Cin, 4 * Cout)


def _packed_conv_weights(w):
    Cout, Cin = w.shape[0], w.shape[1]
    w9 = w.astype(jnp.bfloat16).transpose(2, 3, 1, 0).reshape(
        9, Cin, Cout)
    w10 = jnp.concatenate([w9, jnp.zeros((1, Cin, Cout), jnp.bfloat16)])
    idx = [[dh * 3 + (st - sp) if 0 <= st - sp <= 2 else 9
            for sp in range(2)]
           for dh in range(3) for st in range(4)]
    sel = w10[jnp.array(idx)].transpose(0, 2, 1, 3)
    return sel.reshape(3, 4 * Cin, 2 * Cout)


def _affine_relu(x, sc_ref, sh_ref):
    y = jnp.maximum(x.astype(jnp.float32) * sc_ref[...] + sh_ref[...], 0.0)
    return y.astype(jnp.bfloat16)


def _conv_cols(x):
    z = jnp.zeros_like(x[:, :1])
    xh = jnp.concatenate([z, x, z], axis=1)
    zc = jnp.zeros_like(xh[:, :, :1, :])
    xl = jnp.concatenate([zc, xh[:, :, :-1, :]], axis=2)
    xr = jnp.concatenate([xh[:, :, 1:, :], zc], axis=2)
    return xl, xh, xr


def _parity_matmuls(x, w_ref, H, W, ipb):
    xl, xh, xr = _conv_cols(x)
    cw01 = jnp.concatenate([xl, xh], axis=-1)
    cw12 = jnp.concatenate([xh, xr], axis=-1)
    cws = (cw01, cw12)
    ys = []
    for p, (rh, rw) in enumerate(_PARITIES):
        dhs = (0, 1) if rh == 0 else (1, 2)
        cw = cws[rw]
        acc = None
        for t, dh in enumerate(dhs):
            m = cw[:, dh:dh + H].reshape(ipb * H * W, cw.shape[-1])
            d = jnp.dot(m, w_ref[p, t], preferred_element_type=jnp.float32)
            acc = d if acc is None else acc + d
        ys.append(acc.astype(jnp.bfloat16))
    return ys


def _col9_matmul(x, w_ref, H, W, ipb):
    xl, xh, xr = _conv_cols(x)
    cw = jnp.concatenate([xl, xh, xr], axis=-1)
    acc = None
    for dh in range(3):
        m = cw[:, dh:dh + H].reshape(ipb * H * W, cw.shape[-1])
        d = jnp.dot(m, w_ref[dh], preferred_element_type=jnp.float32)
        acc = d if acc is None else acc + d
    return acc


def _stats4(s_ref, q_ref, ys):
    ss, qs = [], []
    for y in ys:
        yf = y.astype(jnp.float32)
        ss.append(jnp.sum(yf, axis=0, keepdims=True))
        qs.append(jnp.sum(yf * yf, axis=0, keepdims=True))
    s_ref[...] = jnp.concatenate(ss, axis=-1).reshape(s_ref.shape)
    q_ref[...] = jnp.concatenate(qs, axis=-1).reshape(q_ref.shape)


def _store_d2s(o_ref, ys, H, W, ipb):
    C = ys[0].shape[-1]
    for p, (rh, rw) in enumerate(_PARITIES):
        y = ys[p].astype(jnp.float32).reshape(ipb, H, W, C // 128, 128)
        o_ref[:, :, rh, pl.ds(rw, W, 2)] = y


def _store_d2s_packed(o_ref, ys, H, W, ipb):
    C = ys[0].shape[-1]
    for rh in range(2):
        y = jnp.concatenate([ys[2 * rh], ys[2 * rh + 1]], axis=-1)
        o_ref[:, :, rh] = y.reshape(ipb, H, W, 2 * C)


def _packed_cols(x, R, W, ipb):
    C2 = x.shape[-1]
    C = C2 // 2
    z = jnp.zeros_like(x[:, :1])
    xh = jnp.concatenate([z, x, z], axis=1)
    zc = jnp.zeros_like(xh[:, :, :1, :])
    xl = jnp.concatenate([zc, xh[:, :, :-1, :]], axis=2)
    xr = jnp.concatenate([xh[:, :, 1:, :], zc], axis=2)
    return jnp.concatenate(
        [xl[..., C:], xh[..., :C], xh[..., C:], xr[..., :C]], axis=-1)


def _lin_deconv_body(a_ref, wl_ref, bl_ref, w_ref, o_ref, s_ref, q_ref, *,
                     H, W, ipb):
    h = (jnp.dot(a_ref[...], wl_ref[...], preferred_element_type=jnp.float32)
         + bl_ref[...]).astype(jnp.bfloat16)
    x = h.reshape(ipb, H, W, h.shape[-1] // (H * W))
    ys = _parity_matmuls(x, w_ref, H, W, ipb)
    _stats4(s_ref, q_ref, ys)
    _store_d2s(o_ref, ys, H, W, ipb)


def _deconv_body(x_ref, sc_ref, sh_ref, w_ref, o_ref, s_ref, q_ref, *,
                 H, W, ipb):
    x = _affine_relu(x_ref[...], sc_ref, sh_ref)
    ys = _parity_matmuls(x, w_ref, H, W, ipb)
    _stats4(s_ref, q_ref, ys)
    _store_d2s(o_ref, ys, H, W, ipb)


def _deconv9_body(x_ref, sc_ref, sh_ref, w_ref, o_ref, s_ref, q_ref, *,
                  H, W, ipb):
    x = _affine_relu(x_ref[...], sc_ref, sh_ref)
    y = _col9_matmul(x, w_ref, H, W, ipb).astype(jnp.bfloat16)
    C = y.shape[-1] // 4
    ys = [y[:, p * C:(p + 1) * C] for p in range(4)]
    _stats4(s_ref, q_ref, ys)
    _store_d2s_packed(o_ref, ys, H, W, ipb)


def _packed_matmul(x, w_ref, R, W, ipb):
    cw = _packed_cols(x, R, W, ipb)
    acc = None
    for dh in range(3):
        m = cw[:, dh:dh + R].reshape(ipb * R * W, cw.shape[-1])
        d = jnp.dot(m, w_ref[dh], preferred_element_type=jnp.float32)
        acc = d if acc is None else acc + d
    return acc


def _conv_packed_body(x_ref, sc_ref, sh_ref, w_ref, o_ref, s_ref, q_ref, *,
                      H, W, ipb):
    x = _affine_relu(x_ref[...], sc_ref, sh_ref)
    y = _packed_matmul(x, w_ref, H, W, ipb).astype(jnp.bfloat16)
    _stats4(s_ref, q_ref, [y])
    o_ref[...] = y.reshape(ipb, H, W, y.shape[-1])


def _conv_tanh_packed_body(x_ref, sc_ref, sh_ref, w_ref, o_ref, *,
                           H, W, ipb):
    x = _affine_relu(x_ref[...], sc_ref, sh_ref)
    o_ref[...] = jnp.tanh(_packed_matmul(x, w_ref, H, W, ipb))


def _pick_ipb(N, HW):
    ipb = max(1, min(N // 2, 1024 // HW))
    while N % ipb:
        ipb -= 1
    return ipb


def _launch_conv(body, x, scale, shift, wmat, Cout, out_kind):
    N, H, W, C = x.shape
    ipb = _pick_ipb(N, H * W)
    G = N // ipb
    kern = functools.partial(body, H=H, W=W, ipb=ipb)
    in_specs = [
        pl.BlockSpec((ipb, H, W, C), lambda i: (i, 0, 0, 0)),
        pl.BlockSpec((1, C), lambda i: (0, 0)),
        pl.BlockSpec((1, C), lambda i: (0, 0)),
        pl.BlockSpec(wmat.shape, lambda i: (0,) * wmat.ndim),
    ]
    if out_kind == "d2s":
        out_shapes = [jax.ShapeDtypeStruct(
            (N, H, 2, 2 * W, Cout // 128, 128), jnp.float32)]
        out_specs = [pl.BlockSpec((ipb, H, 2, 2 * W, Cout // 128, 128),
                                  lambda i: (i, 0, 0, 0, 0, 0))]
        stat_c = 4 * Cout
    elif out_kind == "d2s_packed":
        out_shapes = [jax.ShapeDtypeStruct((N, H, 2, W, 2 * Cout),
                                           jnp.bfloat16)]
        out_specs = [pl.BlockSpec((ipb, H, 2, W, 2 * Cout),
                                  lambda i: (i, 0, 0, 0, 0))]
        stat_c = 4 * Cout
    elif out_kind == "plane":
        out_shapes = [jax.ShapeDtypeStruct((N, H, W, Cout), jnp.bfloat16)]
        out_specs = [pl.BlockSpec((ipb, H, W, Cout),
                                  lambda i: (i, 0, 0, 0))]
        stat_c = Cout
    else:
        out_shapes = [jax.ShapeDtypeStruct((N * H * W, Cout), jnp.float32)]
        out_specs = [pl.BlockSpec((ipb * H * W, Cout), lambda i: (i, 0))]
        stat_c = 0
    if stat_c:
        out_shapes += [jax.ShapeDtypeStruct((G, 1, stat_c), jnp.float32)] * 2
        out_specs += [pl.BlockSpec((1, 1, stat_c), lambda i: (i, 0, 0))] * 2
    out = pl.pallas_call(
        kern,
        out_shape=tuple(out_shapes),
        grid=(G,),
        in_specs=in_specs,
        out_specs=tuple(out_specs),
        compiler_params=pltpu.CompilerParams(
            dimension_semantics=("parallel",)),
    )(x, scale, shift, wmat)
    if out_kind == "d2s":
        return out[0].reshape(N, 2 * H, 2 * W, Cout), out[1], out[2]
    if out_kind == "d2s_packed":
        return out[0].reshape(N, 2 * H, W, 2 * Cout), out[1], out[2]
    return out if stat_c else out[0]


def kernel(emb, lin_w, lin_b, ct1_w, ct2_w, ct3_w, ct4_w, c5_w, c6_w,
           bn1_g, bn1_b, bn2_g, bn2_b, bn3_g, bn3_b, bn4_g, bn4_b,
           bn5_g, bn5_b, z, labels):
    B = z.shape[0]
    ngf = c5_w.shape[0]
    C0 = 8 * ngf
    nc = c6_w.shape[0]

    a = jnp.concatenate([emb[labels], z.reshape(B, -1)],
                        axis=1).astype(jnp.bfloat16)
    wl = lin_w.reshape(C0, 16, -1).transpose(1, 0, 2).reshape(16 * C0, -1).T
    bl = lin_b.reshape(C0, 16).T.reshape(1, 16 * C0)

    H, W = 4, 4
    Cout = ct1_w.shape[1]
    ipb = _pick_ipb(B, H * W)
    G = B // ipb
    out1 = pl.pallas_call(
        functools.partial(_lin_deconv_body, H=H, W=W, ipb=ipb),
        out_shape=(
            jax.ShapeDtypeStruct((B, H, 2, 2 * W, Cout // 128, 128),
                                 jnp.float32),
            jax.ShapeDtypeStruct((G, 1, 4 * Cout), jnp.float32),
            jax.ShapeDtypeStruct((G, 1, 4 * Cout), jnp.float32),
        ),
        grid=(G,),
        in_specs=[
            pl.BlockSpec((ipb, a.shape[1]), lambda i: (i, 0)),
            pl.BlockSpec(wl.shape, lambda i: (0, 0)),
            pl.BlockSpec((1, 16 * C0), lambda i: (0, 0)),
            pl.BlockSpec((4, 2, 2 * C0, Cout), lambda i: (0, 0, 0, 0)),
        ],
        out_specs=(
            pl.BlockSpec((ipb, H, 2, 2 * W, Cout // 128, 128),
                         lambda i: (i, 0, 0, 0, 0, 0)),
            pl.BlockSpec((1, 1, 4 * Cout), lambda i: (i, 0, 0)),
            pl.BlockSpec((1, 1, 4 * Cout), lambda i: (i, 0, 0)),
        ),
        compiler_params=pltpu.CompilerParams(
            dimension_semantics=("parallel",)),
    )(a, wl.astype(jnp.bfloat16), bl.astype(jnp.float32),
      _parity_weights(ct1_w))
    x = out1[0].reshape(B, 2 * H, 2 * W, Cout)
    s, q = out1[1], out1[2]

    def coeffs(s, q, count, gamma, beta, fold4):
        s = s.sum(axis=(0, 1))
        q = q.sum(axis=(0, 1))
        if fold4:
            Cc = gamma.shape[0]
            s = s.reshape(4, Cc).sum(axis=0)
            q = q.reshape(4, Cc).sum(axis=0)
        sc, sh = _bn_scale_shift(s, q, count, gamma, beta)
        return sc.reshape(1, -1), sh.reshape(1, -1)

    for w_t, (g, b) in ((ct2_w, (bn1_g, bn1_b)), (ct3_w, (bn2_g, bn2_b))):
        N, H2, W2, C = x.shape
        sc, sh = coeffs(s, q, float(N * H2 * W2), g, b, True)
        Cout = w_t.shape[1]
        x, s, q = _launch_conv(
            _deconv_body, x, sc, sh, _parity_weights(w_t), Cout, "d2s")

    N, H2, W2, C = x.shape
    sc, sh = coeffs(s, q, float(N * H2 * W2), bn3_g, bn3_b, True)
    Cout = ct4_w.shape[1]
    x, s, q = _launch_conv(
        _deconv9_body, x, sc, sh, _dense_deconv_weights(ct4_w), Cout,
        "d2s_packed")

    N, R, Wp, C2 = x.shape
    cnt = float(N * R * Wp * 2)
    sc, sh = coeffs(s, q, cnt, bn4_g, bn4_b, True)
    sc = jnp.concatenate([sc, sc], axis=1)
    sh = jnp.concatenate([sh, sh], axis=1)
    x, s, q = _launch_conv(
        _conv_packed_body, x, sc, sh, _packed_conv_weights(c5_w),
        2 * ngf, "plane")

    s = s.sum(axis=(0, 1)).reshape(2, ngf).sum(axis=0)
    q_ = q.sum(axis=(0, 1)).reshape(2, ngf).sum(axis=0)
    sc, sh = _bn_scale_shift(s, q_, cnt, bn5_g, bn5_b)
    sc = jnp.concatenate([sc, sc]).reshape(1, -1)
    sh = jnp.concatenate([sh, sh]).reshape(1, -1)
    out = _launch_conv(
        _conv_tanh_packed_body, x, sc, sh, _packed_conv_weights(c6_w),
        2 * nc, "rows")
    img = out.reshape(N, R, 2 * Wp, nc)
    return img.transpose(0, 3, 1, 2)
